# baseline (speedup 1.0000x reference)
import dataclasses
import functools

import jax
import jax.numpy as jnp
from jax import lax
from jax.experimental import pallas as pl
from jax.experimental.pallas import tpu as pltpu
from jax.experimental.pallas import tpu_sc as plsc

_GAMMA = 24.0
_B = 4096
_NEG = 128
_D = 64
_L = 16
_NV = _D // _L
_NC = 2
_NS = 16
_NW = _NC * _NS
_BPW = _B // _NW
_CB = 4
_CROWS = _CB * _NEG
_NCHUNK = _BPW // _CB


def _sc_body(ent_hbm, rel_hbm, aidx_hbm, ridx_hbm, pidx_hbm, nidx_hbm,
             pos_hbm, neg_hbm,
             aidx_v, ridx_v, pidx_v, arow_v, rrow_v, prow_v, cent_v, pos_v,
             nidx0_v, nidx1_v, g0_v, g1_v, o0_v, o1_v,
             sem_a, sem_r, sem_p, sem0, sem1):
  wid = lax.axis_index("s") * _NC + lax.axis_index("c")
  base = wid * _BPW
  nbase = wid * (_BPW * _NEG)

  pltpu.sync_copy(aidx_hbm.at[pl.ds(base, _BPW)], aidx_v)
  pltpu.sync_copy(ridx_hbm.at[pl.ds(base, _BPW)], ridx_v)
  pltpu.sync_copy(pidx_hbm.at[pl.ds(base, _BPW)], pidx_v)
  ca = pltpu.async_copy(ent_hbm.at[aidx_v], arow_v, sem_a)
  cr = pltpu.async_copy(rel_hbm.at[ridx_v], rrow_v, sem_r)
  cp = pltpu.async_copy(ent_hbm.at[pidx_v], prow_v, sem_p)

  pltpu.sync_copy(nidx_hbm.at[pl.ds(nbase, _CROWS)], nidx0_v)
  pltpu.async_copy(ent_hbm.at[nidx0_v], g0_v, sem0)

  ca.wait()
  cr.wait()
  cp.wait()

  lane = lax.iota(jnp.int32, _L)

  @pl.loop(0, _BPW // _L)
  def _(g):
    out = jnp.zeros((_L,), jnp.float32)
    for jj in range(_L):
      b = g * _L + jj
      acc = None
      for k in range(_NV):
        sl = pl.ds(k * _L, _L)
        c = arow_v[b, sl] + rrow_v[b, sl]
        cent_v[b, sl] = c
        d = jnp.abs(c - prow_v[b, sl])
        acc = d if acc is None else acc + d
      out = jnp.where(lane == jj, jnp.sum(acc), out)
    pos_v[pl.ds(g * _L, _L)] = _GAMMA - out

  pltpu.sync_copy(pos_v, pos_hbm.at[pl.ds(base, _BPW)])

  def start_gather(cidx, idxbuf, gbuf, sem):
    pltpu.sync_copy(nidx_hbm.at[pl.ds(nbase + cidx * _CROWS, _CROWS)], idxbuf)
    pltpu.async_copy(ent_hbm.at[idxbuf], gbuf, sem)

  lane2 = lax.iota(jnp.int32, _L)

  def compute_chunk(cidx, gbuf, obuf):
    for bb in range(_CB):
      brow = cidx * _CB + bb
      cvec = [cent_v[brow, pl.ds(k * _L, _L)] for k in range(_NV)]

      @pl.loop(0, _NEG // _L)
      def _(g):
        out = jnp.zeros((_L,), jnp.float32)
        for jj in range(_L):
          r = bb * _NEG + g * _L + jj
          acc = jnp.abs(cvec[0] - gbuf[r, pl.ds(0, _L)])
          for k in range(1, _NV):
            acc = acc + jnp.abs(cvec[k] - gbuf[r, pl.ds(k * _L, _L)])
          out = jnp.where(lane2 == jj, jnp.sum(acc), out)
        obuf[pl.ds(bb * _NEG + g * _L, _L)] = _GAMMA - out

    pltpu.sync_copy(obuf, neg_hbm.at[pl.ds(nbase + cidx * _CROWS, _CROWS)])

  @pl.loop(0, _NCHUNK, step=2)
  def _(c):
    pltpu.make_async_copy(ent_hbm.at[nidx0_v], g0_v, sem0).wait()
    start_gather(c + 1, nidx1_v, g1_v, sem1)
    compute_chunk(c, g0_v, o0_v)

    pltpu.make_async_copy(ent_hbm.at[nidx1_v], g1_v, sem1).wait()

    @pl.when(c + 2 < _NCHUNK)
    def _():
      start_gather(c + 2, nidx0_v, g0_v, sem0)

    compute_chunk(c + 1, g1_v, o1_v)


def _compiler_params():
  cp = pltpu.CompilerParams()
  fields = pltpu.CompilerParams.__dataclass_fields__
  if "needs_layout_passes" in fields:
    cp = dataclasses.replace(cp, needs_layout_passes=False)
  if "use_tc_tiling_on_sc" in fields:
    cp = dataclasses.replace(cp, use_tc_tiling_on_sc=False)
  return cp


@jax.jit
def _sc_call(entity_embedding, relation_embedding, aidx, ridx, pidx, nidx):
  run = pl.kernel(
      _sc_body,
      compiler_params=_compiler_params(),
      out_type=[
          jax.ShapeDtypeStruct((_B,), jnp.float32),
          jax.ShapeDtypeStruct((_B * _NEG,), jnp.float32),
      ],
      mesh=plsc.VectorSubcoreMesh(core_axis_name="c", subcore_axis_name="s"),
      scratch_types=[
          pltpu.VMEM((_BPW,), jnp.int32),
          pltpu.VMEM((_BPW,), jnp.int32),
          pltpu.VMEM((_BPW,), jnp.int32),
          pltpu.VMEM((_BPW, _D), jnp.float32),
          pltpu.VMEM((_BPW, _D), jnp.float32),
          pltpu.VMEM((_BPW, _D), jnp.float32),
          pltpu.VMEM((_BPW, _D), jnp.float32),
          pltpu.VMEM((_BPW,), jnp.float32),
          pltpu.VMEM((_CROWS,), jnp.int32),
          pltpu.VMEM((_CROWS,), jnp.int32),
          pltpu.VMEM((_CROWS, _D), jnp.float32),
          pltpu.VMEM((_CROWS, _D), jnp.float32),
          pltpu.VMEM((_CROWS,), jnp.float32),
          pltpu.VMEM((_CROWS,), jnp.float32),
          pltpu.SemaphoreType.DMA,
          pltpu.SemaphoreType.DMA,
          pltpu.SemaphoreType.DMA,
          pltpu.SemaphoreType.DMA,
          pltpu.SemaphoreType.DMA,
      ],
  )
  return run(entity_embedding, relation_embedding, aidx, ridx, pidx, nidx)


def kernel(entity_embedding, relation_embedding, subsampling_weight,
           positive_sample, negative_sample, queries):
  aidx = queries[:, 0].astype(jnp.int32)
  ridx = queries[:, 1].astype(jnp.int32)
  pidx = positive_sample.astype(jnp.int32)
  nidx = negative_sample.reshape(-1).astype(jnp.int32)
  pos_logit, neg_flat = _sc_call(
      entity_embedding, relation_embedding, aidx, ridx, pidx, nidx)
  return pos_logit, neg_flat.reshape(_B, _NEG), subsampling_weight
